# Initial kernel scaffold; baseline (speedup 1.0000x reference)
import functools

import jax
import jax.numpy as jnp
from jax import lax
from jax.experimental import pallas as pl
from jax.experimental.pallas import tpu as pltpu
from jax.experimental.pallas import tpu_sc as plsc

N = 10000
NFEAT = 128
NHID = 256
NCLASS = 40
NCLS_PAD = 64
EPS = 1e-5

NC = 2
NS = 16
NW = NC * NS
GROUP = 128
ACC_ROWS = N + 16
ZR = ACC_ROWS // NS
ZR_MAIN = N - (NS - 1) * ZR


def _make_sc_aggregate(d: int, n_groups: int):
    mesh = plsc.VectorSubcoreMesh(core_axis_name="c", subcore_axis_name="s")

    @functools.partial(
        pl.kernel,
        out_type=jax.ShapeDtypeStruct((NC, N, d), jnp.float32),
        mesh=mesh,
        scratch_types=[
            pltpu.VMEM((n_groups, GROUP), jnp.int32),
            pltpu.VMEM((n_groups, GROUP), jnp.int32),
            pltpu.VMEM((GROUP, d), jnp.float32),
            pltpu.VMEM((GROUP, d), jnp.float32),
            pltpu.VMEM_SHARED((ACC_ROWS, d), jnp.float32),
            pltpu.SemaphoreType.DMA,
            pltpu.SemaphoreType.DMA,
        ],
    )
    def agg(table, src_r, dst_r, zeros, out, src_v, dst_v, rows_a, rows_b,
            acc, sem_a, sem_b):
        c = lax.axis_index("c")
        s = lax.axis_index("s")
        wid = s * NC + c

        pltpu.sync_copy(zeros.at[pl.ds(s * ZR, ZR)], acc.at[pl.ds(s * ZR, ZR)])
        pltpu.sync_copy(src_r.at[wid], src_v)
        pltpu.sync_copy(dst_r.at[wid], dst_v)
        plsc.subcore_barrier()

        pltpu.async_copy(table.at[src_v.at[0]], rows_a, sem_a)

        def body(j2, carry):
            j = j2 * 2
            g_b = pltpu.async_copy(table.at[src_v.at[j + 1]], rows_b, sem_b)
            pltpu.make_async_copy(table.at[src_v.at[0]], rows_a, sem_a).wait()
            pltpu.sync_copy(rows_a, acc.at[dst_v.at[j]], add=True)

            @pl.when(j + 2 < n_groups)
            def _():
                pltpu.async_copy(table.at[src_v.at[j + 2]], rows_a, sem_a)

            g_b.wait()
            pltpu.sync_copy(rows_b, acc.at[dst_v.at[j + 1]], add=True)
            return carry

        lax.fori_loop(0, n_groups // 2, body, 0)

        plsc.subcore_barrier()

        base = s * ZR
        pltpu.sync_copy(acc.at[pl.ds(base, ZR_MAIN)],
                        out.at[c].at[pl.ds(base, ZR_MAIN)])

        @pl.when(s < NS - 1)
        def _():
            pltpu.sync_copy(acc.at[pl.ds(base + ZR_MAIN, ZR - ZR_MAIN)],
                            out.at[c].at[pl.ds(base + ZR_MAIN, ZR - ZR_MAIN)])

    return agg


_BM = 400


def _tc1_body(a0, a1, w1, b1, h_ref, sums_ref):
    a = a0[...] + a1[...]
    h = jnp.dot(a, w1[...], preferred_element_type=jnp.float32) + b1[...]
    h_ref[...] = h

    @pl.when(pl.program_id(0) == 0)
    def _():
        sums_ref[...] = jnp.zeros_like(sums_ref)

    sums_ref[0:1, :] += jnp.sum(h, axis=0, keepdims=True)
    sums_ref[1:2, :] += jnp.sum(h * h, axis=0, keepdims=True)


def _tc2_body(h_ref, sums, gamma, beta, w2, y_ref):
    mean = sums[0:1, :] * (1.0 / N)
    var = sums[1:2, :] * (1.0 / N) - mean * mean
    inv = lax.rsqrt(var + EPS)
    hn = (h_ref[...] - mean) * (inv * gamma[...]) + beta[...]
    hr = jnp.maximum(hn, 0.0)
    y_ref[...] = jnp.dot(hr, w2[...], preferred_element_type=jnp.float32)


def _tc3_body(p0, p1, b2, out_ref):
    t = p0[...] + p1[...]
    out_ref[...] = t[:, :NCLASS] + b2[...]


def kernel(x, edge_index, W1, b1, gamma, beta, W2, b2):
    e = edge_index.shape[1]
    n_groups = -(-e // (NW * GROUP))
    if n_groups % 2:
        n_groups += 1
    e_pad = NW * n_groups * GROUP

    src = jnp.concatenate(
        [edge_index[0], jnp.zeros((e_pad - e,), jnp.int32)]).reshape(
            NW, n_groups, GROUP)
    dst = jnp.concatenate(
        [edge_index[1], jnp.full((e_pad - e,), N, jnp.int32)]).reshape(
            NW, n_groups, GROUP)

    zeros1 = jnp.zeros((ACC_ROWS, NFEAT), jnp.float32)
    zeros2 = jnp.zeros((ACC_ROWS, NCLS_PAD), jnp.float32)
    w2p = jnp.pad(W2, ((0, 0), (0, NCLS_PAD - NCLASS)))

    agg1 = _make_sc_aggregate(NFEAT, n_groups)(x, src, dst, zeros1)

    grid = (N // _BM,)
    h, sums = pl.pallas_call(
        _tc1_body,
        grid=grid,
        in_specs=[
            pl.BlockSpec((_BM, NFEAT), lambda i: (i, 0)),
            pl.BlockSpec((_BM, NFEAT), lambda i: (i, 0)),
            pl.BlockSpec((NFEAT, NHID), lambda i: (0, 0)),
            pl.BlockSpec((1, NHID), lambda i: (0, 0)),
        ],
        out_specs=[
            pl.BlockSpec((_BM, NHID), lambda i: (i, 0)),
            pl.BlockSpec((2, NHID), lambda i: (0, 0)),
        ],
        out_shape=[
            jax.ShapeDtypeStruct((N, NHID), jnp.float32),
            jax.ShapeDtypeStruct((2, NHID), jnp.float32),
        ],
    )(agg1[0], agg1[1], W1, b1.reshape(1, NHID))

    y = pl.pallas_call(
        _tc2_body,
        grid=grid,
        in_specs=[
            pl.BlockSpec((_BM, NHID), lambda i: (i, 0)),
            pl.BlockSpec((2, NHID), lambda i: (0, 0)),
            pl.BlockSpec((1, NHID), lambda i: (0, 0)),
            pl.BlockSpec((1, NHID), lambda i: (0, 0)),
            pl.BlockSpec((NHID, NCLS_PAD), lambda i: (0, 0)),
        ],
        out_specs=pl.BlockSpec((_BM, NCLS_PAD), lambda i: (i, 0)),
        out_shape=jax.ShapeDtypeStruct((N, NCLS_PAD), jnp.float32),
    )(h, sums, gamma.reshape(1, NHID), beta.reshape(1, NHID), w2p)

    agg2 = _make_sc_aggregate(NCLS_PAD, n_groups)(y, src, dst, zeros2)

    out = pl.pallas_call(
        _tc3_body,
        grid=grid,
        in_specs=[
            pl.BlockSpec((_BM, NCLS_PAD), lambda i: (i, 0)),
            pl.BlockSpec((_BM, NCLS_PAD), lambda i: (i, 0)),
            pl.BlockSpec((1, NCLASS), lambda i: (0, 0)),
        ],
        out_specs=pl.BlockSpec((_BM, NCLASS), lambda i: (i, 0)),
        out_shape=jax.ShapeDtypeStruct((N, NCLASS), jnp.float32),
    )(agg2[0], agg2[1], b2.reshape(1, NCLASS))

    return out


# speedup vs baseline: 4.6701x; 4.6701x over previous
import functools

import jax
import jax.numpy as jnp
from jax import lax
from jax.experimental import pallas as pl
from jax.experimental.pallas import tpu as pltpu
from jax.experimental.pallas import tpu_sc as plsc

N = 10000
NFEAT = 128
NHID = 256
NCLASS = 40
NCLS_PAD = 64
EPS = 1e-5

NC = 2
NS = 16
NW = NC * NS
GROUP = 128
ACC_ROWS = 10240
ZR = ACC_ROWS // NS
ZR_MAIN = N - (NS - 1) * ZR


def _make_sc_aggregate(d: int, n_groups: int):
    mesh = plsc.VectorSubcoreMesh(core_axis_name="c", subcore_axis_name="s")

    @functools.partial(
        pl.kernel,
        out_type=jax.ShapeDtypeStruct((NC, N, d), jnp.float32),
        mesh=mesh,
        compiler_params=pltpu.CompilerParams(use_tc_tiling_on_sc=False),
        scratch_types=[
            pltpu.VMEM((n_groups // 2, GROUP), jnp.int32),
            pltpu.VMEM((n_groups // 2, GROUP), jnp.int32),
            pltpu.VMEM((GROUP, d), jnp.float32),
            pltpu.VMEM((GROUP, d), jnp.float32),
            pltpu.VMEM_SHARED((ACC_ROWS, d), jnp.float32),
            pltpu.SemaphoreType.DMA,
            pltpu.SemaphoreType.DMA,
        ],
    )
    def agg(table, src_r, dst_r, zeros, out, src_v, dst_v, rows_a, rows_b,
            acc, sem_a, sem_b):
        c = lax.axis_index("c")
        s = lax.axis_index("s")
        wid = s * NC + c
        half = n_groups // 2

        pltpu.sync_copy(zeros.at[pl.ds(s * ZR, ZR)], acc.at[pl.ds(s * ZR, ZR)])
        plsc.subcore_barrier()

        def run_half(hb):
            pltpu.sync_copy(src_r.at[wid].at[pl.ds(hb * half, half)], src_v)
            pltpu.sync_copy(dst_r.at[wid].at[pl.ds(hb * half, half)], dst_v)
            pltpu.async_copy(table.at[src_v.at[0]], rows_a, sem_a)

            def body(j2, carry):
                j = j2 * 2
                g_b = pltpu.async_copy(table.at[src_v.at[j + 1]], rows_b, sem_b)
                pltpu.make_async_copy(table.at[src_v.at[0]], rows_a, sem_a).wait()
                pltpu.sync_copy(rows_a, acc.at[dst_v.at[j]], add=True)

                @pl.when(j + 2 < half)
                def _():
                    pltpu.async_copy(table.at[src_v.at[j + 2]], rows_a, sem_a)

                g_b.wait()
                pltpu.sync_copy(rows_b, acc.at[dst_v.at[j + 1]], add=True)
                return carry

            lax.fori_loop(0, half // 2, body, 0)

        run_half(0)
        run_half(1)

        plsc.subcore_barrier()

        base = s * ZR
        pltpu.sync_copy(acc.at[pl.ds(base, ZR_MAIN)],
                        out.at[c].at[pl.ds(base, ZR_MAIN)])

        @pl.when(s < NS - 1)
        def _():
            pltpu.sync_copy(acc.at[pl.ds(base + ZR_MAIN, ZR - ZR_MAIN)],
                            out.at[c].at[pl.ds(base + ZR_MAIN, ZR - ZR_MAIN)])

    return agg


_BM = 400


def _tc1_body(a0, a1, w1, b1, h_ref, sums_ref):
    a = a0[...] + a1[...]
    h = jnp.dot(a, w1[...], preferred_element_type=jnp.float32) + b1[...]
    h_ref[...] = h

    @pl.when(pl.program_id(0) == 0)
    def _():
        sums_ref[...] = jnp.zeros_like(sums_ref)

    sums_ref[0:1, :] += jnp.sum(h, axis=0, keepdims=True)
    sums_ref[1:2, :] += jnp.sum(h * h, axis=0, keepdims=True)


def _tc2_body(h_ref, sums, gamma, beta, w2, y_ref):
    mean = sums[0:1, :] * (1.0 / N)
    var = sums[1:2, :] * (1.0 / N) - mean * mean
    inv = lax.rsqrt(var + EPS)
    hn = (h_ref[...] - mean) * (inv * gamma[...]) + beta[...]
    hr = jnp.maximum(hn, 0.0)
    y_ref[...] = jnp.dot(hr, w2[...], preferred_element_type=jnp.float32)


def _tc3_body(p0, p1, b2, out_ref):
    t = p0[...] + p1[...]
    out_ref[...] = t[:, :NCLASS] + b2[...]


def kernel(x, edge_index, W1, b1, gamma, beta, W2, b2):
    e = edge_index.shape[1]
    n_groups = -(-e // (NW * GROUP))
    n_groups = -(-n_groups // 4) * 4
    e_pad = NW * n_groups * GROUP

    src = jnp.concatenate(
        [edge_index[0], jnp.zeros((e_pad - e,), jnp.int32)]).reshape(
            NW, n_groups, GROUP)
    dst = jnp.concatenate(
        [edge_index[1], jnp.full((e_pad - e,), N, jnp.int32)]).reshape(
            NW, n_groups, GROUP)

    zeros1 = jnp.zeros((ACC_ROWS, NFEAT), jnp.float32)
    zeros2 = jnp.zeros((ACC_ROWS, NCLS_PAD), jnp.float32)
    w2p = jnp.pad(W2, ((0, 0), (0, NCLS_PAD - NCLASS)))

    agg1 = _make_sc_aggregate(NFEAT, n_groups)(x, src, dst, zeros1)

    grid = (N // _BM,)
    h, sums = pl.pallas_call(
        _tc1_body,
        grid=grid,
        in_specs=[
            pl.BlockSpec((_BM, NFEAT), lambda i: (i, 0)),
            pl.BlockSpec((_BM, NFEAT), lambda i: (i, 0)),
            pl.BlockSpec((NFEAT, NHID), lambda i: (0, 0)),
            pl.BlockSpec((1, NHID), lambda i: (0, 0)),
        ],
        out_specs=[
            pl.BlockSpec((_BM, NHID), lambda i: (i, 0)),
            pl.BlockSpec((2, NHID), lambda i: (0, 0)),
        ],
        out_shape=[
            jax.ShapeDtypeStruct((N, NHID), jnp.float32),
            jax.ShapeDtypeStruct((2, NHID), jnp.float32),
        ],
    )(agg1[0], agg1[1], W1, b1.reshape(1, NHID))

    y = pl.pallas_call(
        _tc2_body,
        grid=grid,
        in_specs=[
            pl.BlockSpec((_BM, NHID), lambda i: (i, 0)),
            pl.BlockSpec((2, NHID), lambda i: (0, 0)),
            pl.BlockSpec((1, NHID), lambda i: (0, 0)),
            pl.BlockSpec((1, NHID), lambda i: (0, 0)),
            pl.BlockSpec((NHID, NCLS_PAD), lambda i: (0, 0)),
        ],
        out_specs=pl.BlockSpec((_BM, NCLS_PAD), lambda i: (i, 0)),
        out_shape=jax.ShapeDtypeStruct((N, NCLS_PAD), jnp.float32),
    )(h, sums, gamma.reshape(1, NHID), beta.reshape(1, NHID), w2p)

    agg2 = _make_sc_aggregate(NCLS_PAD, n_groups)(y, src, dst, zeros2)

    out = pl.pallas_call(
        _tc3_body,
        grid=grid,
        in_specs=[
            pl.BlockSpec((_BM, NCLS_PAD), lambda i: (i, 0)),
            pl.BlockSpec((_BM, NCLS_PAD), lambda i: (i, 0)),
            pl.BlockSpec((1, NCLASS), lambda i: (0, 0)),
        ],
        out_specs=pl.BlockSpec((_BM, NCLASS), lambda i: (i, 0)),
        out_shape=jax.ShapeDtypeStruct((N, NCLASS), jnp.float32),
    )(agg2[0], agg2[1], b2.reshape(1, NCLASS))

    return out


# speedup vs baseline: 4.6729x; 1.0006x over previous
import functools

import jax
import jax.numpy as jnp
from jax import lax
from jax.experimental import pallas as pl
from jax.experimental.pallas import tpu as pltpu
from jax.experimental.pallas import tpu_sc as plsc

N = 10000
NFEAT = 128
NHID = 256
NCLASS = 40
NCLS_PAD = 64
EPS = 1e-5

NC = 2
NS = 16
NW = NC * NS
GROUP = 128
ACC_ROWS = 10240
ZR = ACC_ROWS // NS
ZR_MAIN = N - (NS - 1) * ZR


def _make_sc_aggregate(d: int, n_groups: int):
    mesh = plsc.VectorSubcoreMesh(core_axis_name="c", subcore_axis_name="s")

    @functools.partial(
        pl.kernel,
        out_type=jax.ShapeDtypeStruct((NC, N, d), jnp.float32),
        mesh=mesh,
        compiler_params=pltpu.CompilerParams(use_tc_tiling_on_sc=False),
        scratch_types=[
            pltpu.VMEM((n_groups // 2, GROUP), jnp.int32),
            pltpu.VMEM((n_groups // 2, GROUP), jnp.int32),
            pltpu.VMEM((GROUP, d), jnp.float32),
            pltpu.VMEM((GROUP, d), jnp.float32),
            pltpu.VMEM_SHARED((ACC_ROWS, d), jnp.float32),
            pltpu.SemaphoreType.DMA,
            pltpu.SemaphoreType.DMA,
        ],
    )
    def agg(table, src_r, dst_r, zeros, out, src_v, dst_v, rows_a, rows_b,
            acc, sem_a, sem_b):
        c = lax.axis_index("c")
        s = lax.axis_index("s")
        wid = s * NC + c
        half = n_groups // 2

        pltpu.sync_copy(zeros.at[pl.ds(s * ZR, ZR)], acc.at[pl.ds(s * ZR, ZR)])
        plsc.subcore_barrier()

        def run_half(hb):
            pltpu.sync_copy(src_r.at[wid].at[pl.ds(hb * half, half)], src_v)
            pltpu.sync_copy(dst_r.at[wid].at[pl.ds(hb * half, half)], dst_v)
            pltpu.async_copy(table.at[src_v.at[0]], rows_a, sem_a)

            def body(j2, carry):
                j = j2 * 2
                g_b = pltpu.async_copy(table.at[src_v.at[j + 1]], rows_b, sem_b)
                pltpu.make_async_copy(table.at[src_v.at[0]], rows_a, sem_a).wait()
                pltpu.sync_copy(rows_a, acc.at[dst_v.at[j]], add=True)

                @pl.when(j + 2 < half)
                def _():
                    pltpu.async_copy(table.at[src_v.at[j + 2]], rows_a, sem_a)

                g_b.wait()
                pltpu.sync_copy(rows_b, acc.at[dst_v.at[j + 1]], add=True)
                return carry

            lax.fori_loop(0, half // 2, body, 0)

        run_half(0)
        run_half(1)

        plsc.subcore_barrier()

        base = s * ZR
        pltpu.sync_copy(acc.at[pl.ds(base, ZR_MAIN)],
                        out.at[c].at[pl.ds(base, ZR_MAIN)])

        @pl.when(s < NS - 1)
        def _():
            pltpu.sync_copy(acc.at[pl.ds(base + ZR_MAIN, ZR - ZR_MAIN)],
                            out.at[c].at[pl.ds(base + ZR_MAIN, ZR - ZR_MAIN)])

    return agg


_BM = 400


def _tc1_body(a0, a1, w1, b1, h_ref, sums_ref):
    a = a0[...] + a1[...]
    h = jnp.dot(a, w1[...], preferred_element_type=jnp.float32) + b1[...]
    h_ref[...] = h

    @pl.when(pl.program_id(0) == 0)
    def _():
        sums_ref[...] = jnp.zeros_like(sums_ref)

    sums_ref[0:1, :] += jnp.sum(h, axis=0, keepdims=True)
    sums_ref[1:2, :] += jnp.sum(h * h, axis=0, keepdims=True)


def _tc2_body(h_ref, sums, gamma, beta, w2, y_ref):
    mean = sums[0:1, :] * (1.0 / N)
    var = sums[1:2, :] * (1.0 / N) - mean * mean
    inv = lax.rsqrt(var + EPS)
    hn = (h_ref[...] - mean) * (inv * gamma[...]) + beta[...]
    hr = jnp.maximum(hn, 0.0)
    y_ref[...] = jnp.dot(hr, w2[...], preferred_element_type=jnp.float32)


def _tc3_body(p0, p1, b2, out_ref):
    t = p0[...] + p1[...]
    out_ref[...] = t[:, :NCLASS] + b2[...]


def kernel(x, edge_index, W1, b1, gamma, beta, W2, b2):
    e = edge_index.shape[1]
    n_groups = -(-e // (NW * GROUP))
    n_groups = -(-n_groups // 4) * 4
    e_pad = NW * n_groups * GROUP

    src = jnp.concatenate(
        [edge_index[0], jnp.zeros((e_pad - e,), jnp.int32)]).reshape(
            NW, n_groups, GROUP)
    dummy_dst = N + jnp.arange(e_pad - e, dtype=jnp.int32) % (ACC_ROWS - N)
    dst = jnp.concatenate([edge_index[1], dummy_dst]).reshape(
        NW, n_groups, GROUP)

    zeros1 = jnp.zeros((ACC_ROWS, NFEAT), jnp.float32)
    zeros2 = jnp.zeros((ACC_ROWS, NCLS_PAD), jnp.float32)
    w2p = jnp.pad(W2, ((0, 0), (0, NCLS_PAD - NCLASS)))

    agg1 = _make_sc_aggregate(NFEAT, n_groups)(x, src, dst, zeros1)

    grid = (N // _BM,)
    h, sums = pl.pallas_call(
        _tc1_body,
        grid=grid,
        in_specs=[
            pl.BlockSpec((_BM, NFEAT), lambda i: (i, 0)),
            pl.BlockSpec((_BM, NFEAT), lambda i: (i, 0)),
            pl.BlockSpec((NFEAT, NHID), lambda i: (0, 0)),
            pl.BlockSpec((1, NHID), lambda i: (0, 0)),
        ],
        out_specs=[
            pl.BlockSpec((_BM, NHID), lambda i: (i, 0)),
            pl.BlockSpec((2, NHID), lambda i: (0, 0)),
        ],
        out_shape=[
            jax.ShapeDtypeStruct((N, NHID), jnp.float32),
            jax.ShapeDtypeStruct((2, NHID), jnp.float32),
        ],
    )(agg1[0], agg1[1], W1, b1.reshape(1, NHID))

    y = pl.pallas_call(
        _tc2_body,
        grid=grid,
        in_specs=[
            pl.BlockSpec((_BM, NHID), lambda i: (i, 0)),
            pl.BlockSpec((2, NHID), lambda i: (0, 0)),
            pl.BlockSpec((1, NHID), lambda i: (0, 0)),
            pl.BlockSpec((1, NHID), lambda i: (0, 0)),
            pl.BlockSpec((NHID, NCLS_PAD), lambda i: (0, 0)),
        ],
        out_specs=pl.BlockSpec((_BM, NCLS_PAD), lambda i: (i, 0)),
        out_shape=jax.ShapeDtypeStruct((N, NCLS_PAD), jnp.float32),
    )(h, sums, gamma.reshape(1, NHID), beta.reshape(1, NHID), w2p)

    agg2 = _make_sc_aggregate(NCLS_PAD, n_groups)(y, src, dst, zeros2)

    out = pl.pallas_call(
        _tc3_body,
        grid=grid,
        in_specs=[
            pl.BlockSpec((_BM, NCLS_PAD), lambda i: (i, 0)),
            pl.BlockSpec((_BM, NCLS_PAD), lambda i: (i, 0)),
            pl.BlockSpec((1, NCLASS), lambda i: (0, 0)),
        ],
        out_specs=pl.BlockSpec((_BM, NCLASS), lambda i: (i, 0)),
        out_shape=jax.ShapeDtypeStruct((N, NCLASS), jnp.float32),
    )(agg2[0], agg2[1], b2.reshape(1, NCLASS))

    return out


# speedup vs baseline: 4.9497x; 1.0592x over previous
import functools

import jax
import jax.numpy as jnp
from jax import lax
from jax.experimental import pallas as pl
from jax.experimental.pallas import tpu as pltpu
from jax.experimental.pallas import tpu_sc as plsc

N = 10000
NFEAT = 128
NHID = 256
NCLASS = 40
NCLS_PAD = 64
EPS = 1e-5

NC = 2
NS = 16
NW = NC * NS
GROUP = 128
ACC_ROWS = 10240
ZR = ACC_ROWS // NS
ZR_MAIN = N - (NS - 1) * ZR


CHUNK = 32


def _make_sc_aggregate(d: int, g0: int, g1: int):
    mesh = plsc.VectorSubcoreMesh(core_axis_name="c", subcore_axis_name="s")

    @functools.partial(
        pl.kernel,
        out_type=jax.ShapeDtypeStruct((NC, N, d), jnp.float32),
        mesh=mesh,
        compiler_params=pltpu.CompilerParams(use_tc_tiling_on_sc=False),
        scratch_types=[
            pltpu.VMEM((CHUNK, GROUP), jnp.int32),
            pltpu.VMEM((CHUNK, GROUP), jnp.int32),
            pltpu.VMEM((GROUP, d), jnp.float32),
            pltpu.VMEM((GROUP, d), jnp.float32),
            pltpu.VMEM_SHARED((ACC_ROWS, d), jnp.float32),
            pltpu.SemaphoreType.DMA,
            pltpu.SemaphoreType.DMA,
        ],
    )
    def agg(table, src_r, dst_r, zeros, out, src_v, dst_v, rows_a, rows_b,
            acc, sem_a, sem_b):
        c = lax.axis_index("c")
        s = lax.axis_index("s")

        pltpu.sync_copy(zeros.at[pl.ds(s * ZR, ZR)], acc.at[pl.ds(s * ZR, ZR)])
        plsc.subcore_barrier()

        def run(gbase, g):
            for off in range(0, g, CHUNK):
                cs = min(CHUNK, g - off)
                pltpu.sync_copy(src_r.at[pl.ds(gbase + off, cs)],
                                src_v.at[pl.ds(0, cs)])
                pltpu.sync_copy(dst_r.at[pl.ds(gbase + off, cs)],
                                dst_v.at[pl.ds(0, cs)])
                pltpu.async_copy(table.at[src_v.at[0]], rows_a, sem_a)

                def body(j2, carry, cs=cs):
                    j = j2 * 2
                    g_b = pltpu.async_copy(table.at[src_v.at[j + 1]], rows_b,
                                           sem_b)
                    pltpu.make_async_copy(table.at[src_v.at[0]], rows_a,
                                          sem_a).wait()
                    pltpu.sync_copy(rows_a, acc.at[dst_v.at[j]], add=True)

                    @pl.when(j + 2 < cs)
                    def _():
                        pltpu.async_copy(table.at[src_v.at[j + 2]], rows_a,
                                         sem_a)

                    g_b.wait()
                    pltpu.sync_copy(rows_b, acc.at[dst_v.at[j + 1]], add=True)
                    return carry

                lax.fori_loop(0, cs // 2, body, 0)

        @pl.when(c == 0)
        def _():
            run(s * g0, g0)

        @pl.when(c == 1)
        def _():
            run(NS * g0 + s * g1, g1)

        plsc.subcore_barrier()

        base = s * ZR
        pltpu.sync_copy(acc.at[pl.ds(base, ZR_MAIN)],
                        out.at[c].at[pl.ds(base, ZR_MAIN)])

        @pl.when(s < NS - 1)
        def _():
            pltpu.sync_copy(acc.at[pl.ds(base + ZR_MAIN, ZR - ZR_MAIN)],
                            out.at[c].at[pl.ds(base + ZR_MAIN, ZR - ZR_MAIN)])

    return agg


_BM = 400


def _tc1_body(a0, a1, w1, b1, h_ref, sums_ref):
    a = a0[...] + a1[...]
    h = jnp.dot(a, w1[...], preferred_element_type=jnp.float32) + b1[...]
    h_ref[...] = h

    @pl.when(pl.program_id(0) == 0)
    def _():
        sums_ref[...] = jnp.zeros_like(sums_ref)

    sums_ref[0:1, :] += jnp.sum(h, axis=0, keepdims=True)
    sums_ref[1:2, :] += jnp.sum(h * h, axis=0, keepdims=True)


def _tc2_body(h_ref, sums, gamma, beta, w2, y_ref):
    mean = sums[0:1, :] * (1.0 / N)
    var = sums[1:2, :] * (1.0 / N) - mean * mean
    inv = lax.rsqrt(var + EPS)
    hn = (h_ref[...] - mean) * (inv * gamma[...]) + beta[...]
    hr = jnp.maximum(hn, 0.0)
    y_ref[...] = jnp.dot(hr, w2[...], preferred_element_type=jnp.float32)


def _tc3_body(p0, p1, b2, out_ref):
    t = p0[...] + p1[...]
    out_ref[...] = t[:, :NCLASS] + b2[...]


def kernel(x, edge_index, W1, b1, gamma, beta, W2, b2):
    e = edge_index.shape[1]
    p = -(-(-(-e // GROUP)) // NS)
    p = -(-p // 8) * 8
    g_total = NS * p
    e_pad = g_total * GROUP
    g0_1 = round(p * 0.8 / 8) * 8
    g0_2 = round(p * 0.7 / 8) * 8

    src = jnp.concatenate(
        [edge_index[0], jnp.zeros((e_pad - e,), jnp.int32)]).reshape(
            g_total, GROUP)
    dummy_dst = N + jnp.arange(e_pad - e, dtype=jnp.int32) % (ACC_ROWS - N)
    dst = jnp.concatenate([edge_index[1], dummy_dst]).reshape(
        g_total, GROUP)

    zeros1 = jnp.zeros((ACC_ROWS, NFEAT), jnp.float32)
    zeros2 = jnp.zeros((ACC_ROWS, NCLS_PAD), jnp.float32)
    w2p = jnp.pad(W2, ((0, 0), (0, NCLS_PAD - NCLASS)))

    agg1 = _make_sc_aggregate(NFEAT, g0_1, p - g0_1)(x, src, dst, zeros1)

    grid = (N // _BM,)
    h, sums = pl.pallas_call(
        _tc1_body,
        grid=grid,
        in_specs=[
            pl.BlockSpec((_BM, NFEAT), lambda i: (i, 0)),
            pl.BlockSpec((_BM, NFEAT), lambda i: (i, 0)),
            pl.BlockSpec((NFEAT, NHID), lambda i: (0, 0)),
            pl.BlockSpec((1, NHID), lambda i: (0, 0)),
        ],
        out_specs=[
            pl.BlockSpec((_BM, NHID), lambda i: (i, 0)),
            pl.BlockSpec((2, NHID), lambda i: (0, 0)),
        ],
        out_shape=[
            jax.ShapeDtypeStruct((N, NHID), jnp.float32),
            jax.ShapeDtypeStruct((2, NHID), jnp.float32),
        ],
    )(agg1[0], agg1[1], W1, b1.reshape(1, NHID))

    y = pl.pallas_call(
        _tc2_body,
        grid=grid,
        in_specs=[
            pl.BlockSpec((_BM, NHID), lambda i: (i, 0)),
            pl.BlockSpec((2, NHID), lambda i: (0, 0)),
            pl.BlockSpec((1, NHID), lambda i: (0, 0)),
            pl.BlockSpec((1, NHID), lambda i: (0, 0)),
            pl.BlockSpec((NHID, NCLS_PAD), lambda i: (0, 0)),
        ],
        out_specs=pl.BlockSpec((_BM, NCLS_PAD), lambda i: (i, 0)),
        out_shape=jax.ShapeDtypeStruct((N, NCLS_PAD), jnp.float32),
    )(h, sums, gamma.reshape(1, NHID), beta.reshape(1, NHID), w2p)

    agg2 = _make_sc_aggregate(NCLS_PAD, g0_2, p - g0_2)(y, src, dst, zeros2)

    out = pl.pallas_call(
        _tc3_body,
        grid=grid,
        in_specs=[
            pl.BlockSpec((_BM, NCLS_PAD), lambda i: (i, 0)),
            pl.BlockSpec((_BM, NCLS_PAD), lambda i: (i, 0)),
            pl.BlockSpec((1, NCLASS), lambda i: (0, 0)),
        ],
        out_specs=pl.BlockSpec((_BM, NCLASS), lambda i: (i, 0)),
        out_shape=jax.ShapeDtypeStruct((N, NCLASS), jnp.float32),
    )(agg2[0], agg2[1], b2.reshape(1, NCLASS))

    return out
